# baseline (speedup 1.0000x reference)
import jax
import jax.numpy as jnp
from jax.experimental import pallas as pl
from jax.experimental.pallas import tpu as pltpu

B, N, D, R = 8, 4096, 1024, 100
RP = 128
NSPLIT = 4
TN = N // NSPLIT


def _flash_kernel(n0_ref, n1_ref, n2_ref, n3_ref, edge_ref, q_ref, out_ref):
    q = q_ref[...]
    nbs = []
    logits = []
    for h, nr in enumerate((n0_ref, n1_ref, n2_ref, n3_ref)):
        nb = nr[0].astype(jnp.bfloat16)
        nbs.append(nb)
        l = jax.lax.dot_general(
            nb, q, (((1,), (1,)), ((), ())),
            preferred_element_type=jnp.float32)
        mask = edge_ref[0, h * TN:(h + 1) * TN] == 1
        logits.append(jnp.where(mask, l, jnp.float32(-1e30)))

    m = jnp.max(logits[0], axis=0, keepdims=True)
    for l in logits[1:]:
        m = jnp.maximum(m, jnp.max(l, axis=0, keepdims=True))

    s = jnp.zeros((1, RP), jnp.float32)
    acc = jnp.zeros((RP, D), jnp.float32)
    for l, nb in zip(logits, nbs):
        e = jnp.exp(l - m)
        s = s + jnp.sum(e, axis=0, keepdims=True)
        acc = acc + jax.lax.dot_general(
            e.astype(jnp.bfloat16), nb, (((0,), (0,)), ((), ())),
            preferred_element_type=jnp.float32)
    out_ref[0] = acc / s.T


@jax.jit
def _run(node_feature, edge_weight, q_pad):
    edge3 = edge_weight.reshape(B, N, 1)
    nspec = [
        pl.BlockSpec((1, TN, D), lambda b, h=h: (b, h, 0)) for h in range(NSPLIT)
    ]
    out = pl.pallas_call(
        _flash_kernel,
        grid=(B,),
        in_specs=nspec + [
            pl.BlockSpec((1, N, 1), lambda b: (b, 0, 0)),
            pl.BlockSpec((RP, D), lambda b: (0, 0)),
        ],
        out_specs=pl.BlockSpec((1, RP, D), lambda b: (b, 0, 0)),
        out_shape=jax.ShapeDtypeStruct((B, RP, D), jnp.float32),
        compiler_params=pltpu.CompilerParams(
            dimension_semantics=("parallel",),
        ),
    )(node_feature, node_feature, node_feature, node_feature, edge3, q_pad)
    return out[:, :R, :]


def kernel(node_feature, edge_weight, index, mention_count, relation_label,
           is_train, relation_weight):
    q_pad = jnp.zeros((RP, D), jnp.float32).at[:R].set(relation_weight)
    q_pad = q_pad.astype(jnp.bfloat16)
    return _run(node_feature, edge_weight, q_pad)


# speedup vs baseline: 1.0129x; 1.0129x over previous
import jax
import jax.numpy as jnp
from jax.experimental import pallas as pl
from jax.experimental.pallas import tpu as pltpu

B, N, D, R = 8, 4096, 1024, 100
NSPLIT = 8
TN = N // NSPLIT


def _flash_kernel(*refs):
    n_refs = refs[:NSPLIT]
    edge_ref, q_ref, out_ref = refs[NSPLIT:]
    q = q_ref[...].astype(jnp.bfloat16)
    nbs = []
    logits = []
    for h, nr in enumerate(n_refs):
        nb = nr[0].astype(jnp.bfloat16)
        nbs.append(nb)
        l = jax.lax.dot_general(
            nb, q, (((1,), (1,)), ((), ())),
            preferred_element_type=jnp.float32)
        mask = edge_ref[0, h * TN:(h + 1) * TN] == 1
        logits.append(jnp.where(mask, l, jnp.float32(-1e30)))

    m = jnp.max(logits[0], axis=0, keepdims=True)
    for l in logits[1:]:
        m = jnp.maximum(m, jnp.max(l, axis=0, keepdims=True))

    s = jnp.zeros((1, R), jnp.float32)
    acc = jnp.zeros((R, D), jnp.float32)
    for l, nb in zip(logits, nbs):
        e = jnp.exp(l - m)
        s = s + jnp.sum(e, axis=0, keepdims=True)
        acc = acc + jax.lax.dot_general(
            e.astype(jnp.bfloat16), nb, (((0,), (0,)), ((), ())),
            preferred_element_type=jnp.float32)
    out_ref[0] = acc / s.T


@jax.jit
def _run(node_feature, edge_weight, relation_weight):
    edge3 = edge_weight.reshape(B, N, 1)
    nspec = [
        pl.BlockSpec((1, TN, D), lambda b, h=h: (b, h, 0)) for h in range(NSPLIT)
    ]
    return pl.pallas_call(
        _flash_kernel,
        grid=(B,),
        in_specs=nspec + [
            pl.BlockSpec((1, N, 1), lambda b: (b, 0, 0)),
            pl.BlockSpec((R, D), lambda b: (0, 0)),
        ],
        out_specs=pl.BlockSpec((1, R, D), lambda b: (b, 0, 0)),
        out_shape=jax.ShapeDtypeStruct((B, R, D), jnp.float32),
        compiler_params=pltpu.CompilerParams(
            dimension_semantics=("arbitrary",),
        ),
    )(*([node_feature] * NSPLIT), edge3, relation_weight)


def kernel(node_feature, edge_weight, index, mention_count, relation_label,
           is_train, relation_weight):
    return _run(node_feature, edge_weight, relation_weight)


# speedup vs baseline: 1.0532x; 1.0398x over previous
import jax
import jax.numpy as jnp
from jax.experimental import pallas as pl
from jax.experimental.pallas import tpu as pltpu

B, N, D, R = 8, 4096, 1024, 100
NSPLIT = 16
TN = N // NSPLIT


def _flash_kernel(*refs):
    n_refs = refs[:NSPLIT]
    edge_ref, q_ref, out_ref = refs[NSPLIT:]
    q = q_ref[...].astype(jnp.bfloat16)
    nbs = []
    logits = []
    for h, nr in enumerate(n_refs):
        nb = nr[0].astype(jnp.bfloat16)
        nbs.append(nb)
        l = jax.lax.dot_general(
            nb, q, (((1,), (1,)), ((), ())),
            preferred_element_type=jnp.float32)
        mask = edge_ref[0, h * TN:(h + 1) * TN] == 1
        logits.append(jnp.where(mask, l, jnp.float32(-1e30)))

    m = jnp.max(logits[0], axis=0, keepdims=True)
    for l in logits[1:]:
        m = jnp.maximum(m, jnp.max(l, axis=0, keepdims=True))

    s = jnp.zeros((1, R), jnp.float32)
    acc = jnp.zeros((R, D), jnp.float32)
    for l, nb in zip(logits, nbs):
        e = jnp.exp(l - m)
        s = s + jnp.sum(e, axis=0, keepdims=True)
        acc = acc + jax.lax.dot_general(
            e.astype(jnp.bfloat16), nb, (((0,), (0,)), ((), ())),
            preferred_element_type=jnp.float32)
    out_ref[0] = acc / s.T


@jax.jit
def _run(node_feature, edge_weight, relation_weight):
    edge3 = edge_weight.reshape(B, N, 1)
    nspec = [
        pl.BlockSpec((1, TN, D), lambda b, h=h: (b, h, 0)) for h in range(NSPLIT)
    ]
    return pl.pallas_call(
        _flash_kernel,
        grid=(B,),
        in_specs=nspec + [
            pl.BlockSpec((1, N, 1), lambda b: (b, 0, 0)),
            pl.BlockSpec((R, D), lambda b: (0, 0)),
        ],
        out_specs=pl.BlockSpec((1, R, D), lambda b: (b, 0, 0)),
        out_shape=jax.ShapeDtypeStruct((B, R, D), jnp.float32),
        compiler_params=pltpu.CompilerParams(
            dimension_semantics=("arbitrary",),
        ),
    )(*([node_feature] * NSPLIT), edge3, relation_weight)


def kernel(node_feature, edge_weight, index, mention_count, relation_label,
           is_train, relation_weight):
    return _run(node_feature, edge_weight, relation_weight)
